# Initial kernel scaffold; baseline (speedup 1.0000x reference)
import functools

import jax
import jax.numpy as jnp
from jax.experimental import pallas as pl

_PREC = jax.lax.Precision.HIGHEST
_KSEL = 50
_BISECT_ITERS = 28


def _rowsum_body(a_ref, o_ref):
    o_ref[...] = jnp.sum(a_ref[...], axis=1, keepdims=True)


def _rowsum(a, bm=400):
    n, m = a.shape
    return pl.pallas_call(
        _rowsum_body,
        grid=(n // bm,),
        in_specs=[pl.BlockSpec((bm, m), lambda i: (i, 0))],
        out_specs=pl.BlockSpec((bm, 1), lambda i: (i, 0)),
        out_shape=jax.ShapeDtypeStruct((n, 1), a.dtype),
    )(a)


def _scale_body(r_ref, x_ref, w_ref, o_ref):
    o_ref[...] = r_ref[...] * x_ref[...] * w_ref[...]


def _scale_rows(r, x, w, bm=2000):
    n, f = x.shape
    return pl.pallas_call(
        _scale_body,
        grid=(n // bm,),
        in_specs=[
            pl.BlockSpec((bm, 1), lambda i: (i, 0)),
            pl.BlockSpec((bm, f), lambda i: (i, 0)),
            pl.BlockSpec((1, f), lambda i: (0, 0)),
        ],
        out_specs=pl.BlockSpec((bm, f), lambda i: (i, 0)),
        out_shape=jax.ShapeDtypeStruct((n, f), x.dtype),
    )(r, x, w)


def _mm_body(a_ref, c_ref, s_ref, b_ref, o_ref, *, mode):
    y = jax.lax.dot_general(
        a_ref[...], c_ref[...], (((1,), (0,)), ((), ())),
        preferred_element_type=jnp.float32, precision=_PREC)
    y = s_ref[...] * y
    if mode == "norm":
        nrm = jnp.sqrt(jnp.sum(y * y, axis=1, keepdims=True))
        y = y / jnp.maximum(nrm, 1e-12)
    elif mode == "relu_bias":
        y = jnp.maximum(y + b_ref[...], 0.0)
    elif mode == "bias":
        y = y + b_ref[...]
    o_ref[...] = y


def _mm(a, c, s, b, mode, bm):
    n, k = a.shape
    _, f = c.shape
    return pl.pallas_call(
        functools.partial(_mm_body, mode=mode),
        grid=(n // bm,),
        in_specs=[
            pl.BlockSpec((bm, k), lambda i: (i, 0)),
            pl.BlockSpec((k, f), lambda i: (0, 0)),
            pl.BlockSpec((bm, 1), lambda i: (i, 0)),
            pl.BlockSpec((1, f), lambda i: (0, 0)),
        ],
        out_specs=pl.BlockSpec((bm, f), lambda i: (i, 0)),
        out_shape=jax.ShapeDtypeStruct((n, f), jnp.float32),
    )(a, c, s, b)


def _thresh_body(e_ref, et_ref, o_ref, *, k, iters):
    s = jax.lax.dot_general(
        e_ref[...], et_ref[...], (((1,), (0,)), ((), ())),
        preferred_element_type=jnp.float32, precision=_PREC)
    bm = s.shape[0]

    def body(_, carry):
        lo, hi = carry
        mid = 0.5 * (lo + hi)
        cnt = jnp.sum(jnp.where(s >= mid, 1.0, 0.0), axis=1, keepdims=True)
        ge = cnt >= k
        return jnp.where(ge, mid, lo), jnp.where(ge, hi, mid)

    lo = jnp.full((bm, 1), -1.5, jnp.float32)
    hi = jnp.full((bm, 1), 1.5, jnp.float32)
    lo, hi = jax.lax.fori_loop(0, iters, body, (lo, hi))
    o_ref[...] = lo


def _thresh(emb, embT, bm=200):
    n, f = emb.shape
    return pl.pallas_call(
        functools.partial(_thresh_body, k=float(_KSEL), iters=_BISECT_ITERS),
        grid=(n // bm,),
        in_specs=[
            pl.BlockSpec((bm, f), lambda i: (i, 0)),
            pl.BlockSpec((f, n), lambda i: (0, 0)),
        ],
        out_specs=pl.BlockSpec((bm, 1), lambda i: (i, 0)),
        out_shape=jax.ShapeDtypeStruct((n, 1), jnp.float32),
    )(emb, embT)


def _fuse_body(e_ref, et_ref, ti_ref, tj_ref, adj_ref, an_ref, af_ref, d2_ref):
    j = pl.program_id(1)
    s = jax.lax.dot_general(
        e_ref[...], et_ref[...], (((1,), (0,)), ((), ())),
        preferred_element_type=jnp.float32, precision=_PREC)
    sel = ((s >= ti_ref[...]).astype(jnp.float32)
           + (s >= tj_ref[...]).astype(jnp.float32))
    an = 0.5 * s * sel
    af = an + adj_ref[...]
    an_ref[...] = an
    af_ref[...] = af

    @pl.when(j == 0)
    def _():
        d2_ref[...] = jnp.zeros_like(d2_ref)

    d2_ref[...] += jnp.sum(af, axis=1, keepdims=True)


def _fuse(emb, embT, t, tT, adj, bm=400, bn=400):
    n, f = emb.shape
    an, af, d2 = pl.pallas_call(
        _fuse_body,
        grid=(n // bm, n // bn),
        in_specs=[
            pl.BlockSpec((bm, f), lambda i, j: (i, 0)),
            pl.BlockSpec((f, bn), lambda i, j: (0, j)),
            pl.BlockSpec((bm, 1), lambda i, j: (i, 0)),
            pl.BlockSpec((1, bn), lambda i, j: (0, j)),
            pl.BlockSpec((bm, bn), lambda i, j: (i, j)),
        ],
        out_specs=[
            pl.BlockSpec((bm, bn), lambda i, j: (i, j)),
            pl.BlockSpec((bm, bn), lambda i, j: (i, j)),
            pl.BlockSpec((bm, 1), lambda i, j: (i, 0)),
        ],
        out_shape=[
            jax.ShapeDtypeStruct((n, n), jnp.float32),
            jax.ShapeDtypeStruct((n, n), jnp.float32),
            jax.ShapeDtypeStruct((n, 1), jnp.float32),
        ],
    )(emb, embT, t, tT, adj)
    return an, af, d2


def kernel(input, Adj, w_diag1, w_diag2, W1, b1, W2, b2):
    x = input
    n, feat = x.shape
    hdim = W1.shape[1]
    cdim = W2.shape[1]
    zf = jnp.zeros((1, feat), jnp.float32)
    zh = jnp.zeros((1, hdim), jnp.float32)
    zc = jnp.zeros((1, cdim), jnp.float32)

    deg = _rowsum(Adj)
    dinv = jnp.where(deg > 0, jax.lax.rsqrt(deg), 0.0)
    c1 = _scale_rows(dinv, x, w_diag1[None, :])
    h = _mm(Adj, c1, dinv, zf, "scale", bm=200)
    c2 = _scale_rows(dinv, h, w_diag2[None, :])
    emb = _mm(Adj, c2, dinv, zf, "norm", bm=200)
    embT = emb.T

    t = _thresh(emb, embT)
    tT = t.reshape(1, n)

    an, af, d2 = _fuse(emb, embT, t, tT, Adj)
    dinv2 = jnp.where(d2 > 0, jax.lax.rsqrt(d2), 0.0)

    c3 = _mm(x, W1, dinv2, zh, "scale", bm=2000)
    h1 = _mm(af, c3, dinv2, b1[None, :], "relu_bias", bm=200)
    c4 = _mm(h1, W2, dinv2, zc, "scale", bm=2000)
    out = _mm(af, c4, dinv2, b2[None, :], "bias", bm=200)
    return out, an, af


# speedup vs baseline: 9.5457x; 9.5457x over previous
import functools

import jax
import jax.numpy as jnp
from jax.experimental import pallas as pl
from jax.experimental.pallas import tpu as pltpu

_KSEL = 50


def _scale_body(x_ref, w_ref, o_ref):
    o_ref[...] = x_ref[...] * w_ref[...]


def _scale_cols(x, w, bm=2000):
    n, f = x.shape
    bm = min(bm, n)
    return pl.pallas_call(
        _scale_body,
        grid=(n // bm,),
        in_specs=[
            pl.BlockSpec((bm, f), lambda i: (i, 0)),
            pl.BlockSpec((1, f), lambda i: (0, 0)),
        ],
        out_specs=pl.BlockSpec((bm, f), lambda i: (i, 0)),
        out_shape=jax.ShapeDtypeStruct((n, f), x.dtype),
    )(x, w)


def _gcn_body(a_ref, di_ref, dj_ref, c_ref, b_ref, o_ref, *, mode):
    na = di_ref[...] * a_ref[...] * dj_ref[...]
    y = jax.lax.dot_general(
        na, c_ref[...], (((1,), (0,)), ((), ())),
        preferred_element_type=jnp.float32)
    if mode == "relu_bias":
        y = jnp.maximum(y + b_ref[...], 0.0)
    elif mode == "bias":
        y = y + b_ref[...]
    o_ref[...] = y


def _gcn_mm(a, di, dj, c, b, mode, bm=200):
    n, k = a.shape
    _, f = c.shape
    bm = min(bm, n)
    return pl.pallas_call(
        functools.partial(_gcn_body, mode=mode),
        grid=(n // bm,),
        in_specs=[
            pl.BlockSpec((bm, k), lambda i: (i, 0)),
            pl.BlockSpec((bm, 1), lambda i: (i, 0)),
            pl.BlockSpec((1, k), lambda i: (0, 0)),
            pl.BlockSpec((k, f), lambda i: (0, 0)),
            pl.BlockSpec((1, f), lambda i: (0, 0)),
        ],
        out_specs=pl.BlockSpec((bm, f), lambda i: (i, 0)),
        out_shape=jax.ShapeDtypeStruct((n, f), jnp.float32),
    )(a, di, dj, c, b)


def _dot_body(a_ref, c_ref, o_ref):
    o_ref[...] = jax.lax.dot_general(
        a_ref[...], c_ref[...], (((1,), (0,)), ((), ())),
        preferred_element_type=jnp.float32)


def _dot(a, c, bm=2000):
    n, k = a.shape
    _, f = c.shape
    bm = min(bm, n)
    return pl.pallas_call(
        _dot_body,
        grid=(n // bm,),
        in_specs=[
            pl.BlockSpec((bm, k), lambda i: (i, 0)),
            pl.BlockSpec((k, f), lambda i: (0, 0)),
        ],
        out_specs=pl.BlockSpec((bm, f), lambda i: (i, 0)),
        out_shape=jax.ShapeDtypeStruct((n, f), jnp.float32),
    )(a, c)


def _f32_to_ord(b):
    return b ^ (jax.lax.shift_right_arithmetic(b, 31) & jnp.int32(0x7FFFFFFF))


def _thresh_body(ei_ref, e_ref, sim_ref, t_ref, r_ref, *, k, half):
    ei = ei_ref[...]
    e = e_ref[...]
    s1 = jax.lax.dot_general(
        ei[:, :half], e[:, :half], (((1,), (1,)), ((), ())),
        preferred_element_type=jnp.float32)
    s2 = jax.lax.dot_general(
        ei[:, half:], e[:, half:], (((1,), (1,)), ((), ())),
        preferred_element_type=jnp.float32)
    s = s1 + s2
    sim_ref[...] = s
    sc = jnp.where(s == 0.0, 0.0, s)
    si = _f32_to_ord(jax.lax.bitcast_convert_type(sc, jnp.int32))
    bm = s.shape[0]

    def body(_, carry):
        lo, hi = carry
        mid = (jax.lax.shift_right_arithmetic(lo, 1)
               + jax.lax.shift_right_arithmetic(hi, 1)
               + (lo & hi & jnp.int32(1)))
        cnt = jnp.sum(jnp.where(si >= mid, 1.0, 0.0), axis=1, keepdims=True)
        ge = cnt >= k
        return jnp.where(ge, mid, lo), jnp.where(ge, hi, mid)

    lo = jnp.full((bm, 1), jnp.iinfo(jnp.int32).min, jnp.int32)
    hi = jnp.full((bm, 1), jnp.iinfo(jnp.int32).max, jnp.int32)
    lo, hi = jax.lax.fori_loop(0, 32, body, (lo, hi))
    c_gt = jnp.sum(jnp.where(si > lo, 1.0, 0.0), axis=1, keepdims=True)
    t_ref[...] = jax.lax.bitcast_convert_type(_f32_to_ord(lo), jnp.float32)
    r_ref[...] = k - c_gt


def _thresh(emb, bm=200):
    n, f = emb.shape
    bm = min(bm, n)
    return pl.pallas_call(
        functools.partial(_thresh_body, k=float(_KSEL), half=f // 2),
        grid=(n // bm,),
        in_specs=[
            pl.BlockSpec((bm, f), lambda i: (i, 0)),
            pl.BlockSpec((n, f), lambda i: (0, 0)),
        ],
        out_specs=[
            pl.BlockSpec((bm, n), lambda i: (i, 0)),
            pl.BlockSpec((bm, 1), lambda i: (i, 0)),
            pl.BlockSpec((bm, 1), lambda i: (i, 0)),
        ],
        out_shape=[
            jax.ShapeDtypeStruct((n, n), jnp.float32),
            jax.ShapeDtypeStruct((n, 1), jnp.float32),
            jax.ShapeDtypeStruct((n, 1), jnp.float32),
        ],
    )(emb, emb)


def _cumsum_shift(x, axis):
    n = x.shape[axis]
    d = 1
    while d < n:
        if axis == 1:
            pad = jnp.zeros((x.shape[0], d), x.dtype)
            x = x + jnp.concatenate([pad, x[:, : n - d]], axis=1)
        else:
            pad = jnp.zeros((d, x.shape[1]), x.dtype)
            x = x + jnp.concatenate([pad, x[: n - d, :]], axis=0)
        d *= 2
    return x


def _fuse_body(s_ref, ti_ref, ri_ref, tj_ref, rj_ref, adj_ref,
               an_ref, af_ref, d2_ref, vc_ref, hc_ref, *, bn, n):
    i = pl.program_id(0)
    j = pl.program_id(1)
    s = s_ref[...]
    ti = ti_ref[...]
    tj = tj_ref[...]
    col = jax.lax.broadcasted_iota(jnp.int32, s.shape, 1)
    valid = (j * bn + col) < n
    vf = jnp.where(valid, 1.0, 0.0)

    @pl.when(j == 0)
    def _():
        hc_ref[...] = jnp.zeros_like(hc_ref)

    @pl.when(i == 0)
    def _():
        vc_ref[:, pl.ds(j * bn, bn)] = jnp.zeros_like(vc_ref[:, pl.ds(j * bn, bn)])

    eq_i = jnp.where(s == ti, 1.0, 0.0) * vf
    h_rank = hc_ref[...] + _cumsum_shift(eq_i, axis=1)
    sel_i = jnp.where(s > ti, 1.0, 0.0) + eq_i * jnp.where(h_rank <= ri_ref[...], 1.0, 0.0)
    hc_ref[...] += jnp.sum(eq_i, axis=1, keepdims=True)

    eq_j = jnp.where(s == tj, 1.0, 0.0) * vf
    v_rank = vc_ref[:, pl.ds(j * bn, bn)] + _cumsum_shift(eq_j, axis=0)
    sel_j = jnp.where(s > tj, 1.0, 0.0) + eq_j * jnp.where(v_rank <= rj_ref[...], 1.0, 0.0)
    vc_ref[:, pl.ds(j * bn, bn)] += jnp.sum(eq_j, axis=0, keepdims=True)

    an = 0.5 * (s * sel_i + s * sel_j)
    af = an + adj_ref[...]
    an_ref[...] = an
    af_ref[...] = af

    @pl.when(j == 0)
    def _():
        d2_ref[...] = jnp.zeros_like(d2_ref)

    d2_ref[...] += jnp.sum(jnp.where(valid, af, 0.0), axis=1, keepdims=True)


def _fuse(sim, t, r, tT, rT, adj, bm=400, bn=512):
    n = adj.shape[0]
    bm = min(bm, n)
    bn = min(bn, n)
    nj = pl.cdiv(n, bn)
    an, af, d2 = pl.pallas_call(
        functools.partial(_fuse_body, bn=bn, n=n),
        grid=(n // bm, nj),
        in_specs=[
            pl.BlockSpec((bm, bn), lambda i, j: (i, j)),
            pl.BlockSpec((bm, 1), lambda i, j: (i, 0)),
            pl.BlockSpec((bm, 1), lambda i, j: (i, 0)),
            pl.BlockSpec((1, bn), lambda i, j: (0, j)),
            pl.BlockSpec((1, bn), lambda i, j: (0, j)),
            pl.BlockSpec((bm, bn), lambda i, j: (i, j)),
        ],
        out_specs=[
            pl.BlockSpec((bm, bn), lambda i, j: (i, j)),
            pl.BlockSpec((bm, bn), lambda i, j: (i, j)),
            pl.BlockSpec((bm, 1), lambda i, j: (i, 0)),
        ],
        out_shape=[
            jax.ShapeDtypeStruct((n, n), jnp.float32),
            jax.ShapeDtypeStruct((n, n), jnp.float32),
            jax.ShapeDtypeStruct((n, 1), jnp.float32),
        ],
        scratch_shapes=[
            pltpu.VMEM((1, nj * bn), jnp.float32),
            pltpu.VMEM((bm, 1), jnp.float32),
        ],
    )(sim, t, r, tT, rT, adj)
    return an, af, d2


def kernel(input, Adj, w_diag1, w_diag2, W1, b1, W2, b2):
    x = input
    n, feat = x.shape
    zf = jnp.zeros((1, feat), jnp.float32)

    deg = jnp.sum(Adj, axis=1)
    dinv = jnp.where(deg > 0, deg ** -0.5, 0.0)
    di = dinv[:, None]
    dj = dinv[None, :]
    c1 = _scale_cols(x, w_diag1[None, :])
    h = _gcn_mm(Adj, di, dj, c1, zf, "plain")
    c2 = _scale_cols(h, w_diag2[None, :])
    h2 = _gcn_mm(Adj, di, dj, c2, zf, "plain")
    nn = jnp.linalg.norm(h2, axis=1, keepdims=True)
    emb = h2 / jnp.maximum(nn, 1e-12)

    sim, t, r = _thresh(emb)
    tT = t.reshape(1, n)
    rT = r.reshape(1, n)

    an, af, d2 = _fuse(sim, t, r, tT, rT, Adj)
    dinv2 = jnp.where(d2[:, 0] > 0, d2[:, 0] ** -0.5, 0.0)
    di2 = dinv2[:, None]
    dj2 = dinv2[None, :]

    c3 = _dot(x, W1)
    h1 = _gcn_mm(af, di2, dj2, c3, b1[None, :], "relu_bias")
    c4 = _dot(h1, W2)
    out = _gcn_mm(af, di2, dj2, c4, b2[None, :], "bias")
    return out, an, af


# speedup vs baseline: 11.5588x; 1.2109x over previous
import functools

import jax
import jax.numpy as jnp
from jax.experimental import pallas as pl
from jax.experimental.pallas import tpu as pltpu

_KSEL = 50


def _scale_body(x_ref, w_ref, o_ref):
    o_ref[...] = x_ref[...] * w_ref[...]


def _scale_cols(x, w, bm=2000):
    n, f = x.shape
    bm = min(bm, n)
    return pl.pallas_call(
        _scale_body,
        grid=(n // bm,),
        in_specs=[
            pl.BlockSpec((bm, f), lambda i: (i, 0)),
            pl.BlockSpec((1, f), lambda i: (0, 0)),
        ],
        out_specs=pl.BlockSpec((bm, f), lambda i: (i, 0)),
        out_shape=jax.ShapeDtypeStruct((n, f), x.dtype),
    )(x, w)


def _gcn_body(a_ref, di_ref, dj_ref, c_ref, b_ref, o_ref, *, mode):
    na = di_ref[...] * a_ref[...] * dj_ref[...]
    y = jax.lax.dot_general(
        na, c_ref[...], (((1,), (0,)), ((), ())),
        preferred_element_type=jnp.float32)
    if mode == "relu_bias":
        y = jnp.maximum(y + b_ref[...], 0.0)
    elif mode == "bias":
        y = y + b_ref[...]
    o_ref[...] = y


def _gcn_mm(a, di, dj, c, b, mode, bm=200):
    n, k = a.shape
    _, f = c.shape
    bm = min(bm, n)
    return pl.pallas_call(
        functools.partial(_gcn_body, mode=mode),
        grid=(n // bm,),
        in_specs=[
            pl.BlockSpec((bm, k), lambda i: (i, 0)),
            pl.BlockSpec((bm, 1), lambda i: (i, 0)),
            pl.BlockSpec((1, k), lambda i: (0, 0)),
            pl.BlockSpec((k, f), lambda i: (0, 0)),
            pl.BlockSpec((1, f), lambda i: (0, 0)),
        ],
        out_specs=pl.BlockSpec((bm, f), lambda i: (i, 0)),
        out_shape=jax.ShapeDtypeStruct((n, f), jnp.float32),
    )(a, di, dj, c, b)


def _dot_body(a_ref, c_ref, o_ref):
    o_ref[...] = jax.lax.dot_general(
        a_ref[...], c_ref[...], (((1,), (0,)), ((), ())),
        preferred_element_type=jnp.float32)


def _dot(a, c, bm=2000):
    n, k = a.shape
    _, f = c.shape
    bm = min(bm, n)
    return pl.pallas_call(
        _dot_body,
        grid=(n // bm,),
        in_specs=[
            pl.BlockSpec((bm, k), lambda i: (i, 0)),
            pl.BlockSpec((k, f), lambda i: (0, 0)),
        ],
        out_specs=pl.BlockSpec((bm, f), lambda i: (i, 0)),
        out_shape=jax.ShapeDtypeStruct((n, f), jnp.float32),
    )(a, c)


def _f32_to_ord(b):
    return b ^ (jax.lax.shift_right_arithmetic(b, 31) & jnp.int32(0x7FFFFFFF))


def _thresh_body(ei_ref, e_ref, sim_ref, t_ref, r_ref, *, k, half):
    ei = ei_ref[...]
    e = e_ref[...]
    s1 = jax.lax.dot_general(
        ei[:, :half], e[:, :half], (((1,), (1,)), ((), ())),
        preferred_element_type=jnp.float32)
    s2 = jax.lax.dot_general(
        ei[:, half:], e[:, half:], (((1,), (1,)), ((), ())),
        preferred_element_type=jnp.float32)
    s = s1 + s2
    sim_ref[...] = s
    sc = jnp.where(s == 0.0, 0.0, s)
    si = _f32_to_ord(jax.lax.bitcast_convert_type(sc, jnp.int32))
    bm = s.shape[0]

    def cond(carry):
        it, lo, hi = carry
        return (it < 32) & jnp.any(lo + 1 < hi)

    def body(carry):
        it, lo, hi = carry
        mid = (jax.lax.shift_right_arithmetic(lo, 1)
               + jax.lax.shift_right_arithmetic(hi, 1)
               + (lo & hi & jnp.int32(1)))
        cnt = jnp.sum(jnp.where(si >= mid, 1.0, 0.0), axis=1, keepdims=True)
        ge = cnt >= k
        return it + 1, jnp.where(ge, mid, lo), jnp.where(ge, hi, mid)

    lo = jnp.min(si, axis=1, keepdims=True)
    hi = jnp.max(si, axis=1, keepdims=True) + 1
    _, lo, hi = jax.lax.while_loop(cond, body, (jnp.int32(0), lo, hi))
    c_gt = jnp.sum(jnp.where(si > lo, 1.0, 0.0), axis=1, keepdims=True)
    t_ref[...] = jax.lax.bitcast_convert_type(_f32_to_ord(lo), jnp.float32)
    r_ref[...] = k - c_gt


def _thresh(emb, bm=200):
    n, f = emb.shape
    bm = min(bm, n)
    return pl.pallas_call(
        functools.partial(_thresh_body, k=float(_KSEL), half=f // 2),
        grid=(n // bm,),
        in_specs=[
            pl.BlockSpec((bm, f), lambda i: (i, 0)),
            pl.BlockSpec((n, f), lambda i: (0, 0)),
        ],
        out_specs=[
            pl.BlockSpec((bm, n), lambda i: (i, 0)),
            pl.BlockSpec((bm, 1), lambda i: (i, 0)),
            pl.BlockSpec((bm, 1), lambda i: (i, 0)),
        ],
        out_shape=[
            jax.ShapeDtypeStruct((n, n), jnp.float32),
            jax.ShapeDtypeStruct((n, 1), jnp.float32),
            jax.ShapeDtypeStruct((n, 1), jnp.float32),
        ],
    )(emb, emb)


def _cumsum_shift(x, axis):
    n = x.shape[axis]
    d = 1
    while d < n:
        if axis == 1:
            pad = jnp.zeros((x.shape[0], d), x.dtype)
            x = x + jnp.concatenate([pad, x[:, : n - d]], axis=1)
        else:
            pad = jnp.zeros((d, x.shape[1]), x.dtype)
            x = x + jnp.concatenate([pad, x[: n - d, :]], axis=0)
        d *= 2
    return x


def _fuse_body(s_ref, ti_ref, ri_ref, tj_ref, rj_ref, adj_ref,
               an_ref, af_ref, d2_ref, vc_ref, hc_ref, *, bn, n):
    i = pl.program_id(0)
    j = pl.program_id(1)
    s = s_ref[...]
    ti = ti_ref[...]
    tj = tj_ref[...]
    col = jax.lax.broadcasted_iota(jnp.int32, s.shape, 1)
    valid = (j * bn + col) < n
    vf = jnp.where(valid, 1.0, 0.0)

    @pl.when(j == 0)
    def _():
        hc_ref[...] = jnp.zeros_like(hc_ref)

    @pl.when(i == 0)
    def _():
        vc_ref[:, pl.ds(j * bn, bn)] = jnp.zeros_like(vc_ref[:, pl.ds(j * bn, bn)])

    eq_i = jnp.where(s == ti, 1.0, 0.0) * vf
    h_rank = hc_ref[...] + _cumsum_shift(eq_i, axis=1)
    sel_i = jnp.where(s > ti, 1.0, 0.0) + eq_i * jnp.where(h_rank <= ri_ref[...], 1.0, 0.0)
    hc_ref[...] += jnp.sum(eq_i, axis=1, keepdims=True)

    eq_j = jnp.where(s == tj, 1.0, 0.0) * vf
    v_rank = vc_ref[:, pl.ds(j * bn, bn)] + _cumsum_shift(eq_j, axis=0)
    sel_j = jnp.where(s > tj, 1.0, 0.0) + eq_j * jnp.where(v_rank <= rj_ref[...], 1.0, 0.0)
    vc_ref[:, pl.ds(j * bn, bn)] += jnp.sum(eq_j, axis=0, keepdims=True)

    an = 0.5 * (s * sel_i + s * sel_j)
    af = an + adj_ref[...]
    an_ref[...] = an
    af_ref[...] = af

    @pl.when(j == 0)
    def _():
        d2_ref[...] = jnp.zeros_like(d2_ref)

    d2_ref[...] += jnp.sum(jnp.where(valid, af, 0.0), axis=1, keepdims=True)


def _fuse(sim, t, r, tT, rT, adj, bm=400, bn=512):
    n = adj.shape[0]
    bm = min(bm, n)
    bn = min(bn, n)
    nj = pl.cdiv(n, bn)
    an, af, d2 = pl.pallas_call(
        functools.partial(_fuse_body, bn=bn, n=n),
        grid=(n // bm, nj),
        in_specs=[
            pl.BlockSpec((bm, bn), lambda i, j: (i, j)),
            pl.BlockSpec((bm, 1), lambda i, j: (i, 0)),
            pl.BlockSpec((bm, 1), lambda i, j: (i, 0)),
            pl.BlockSpec((1, bn), lambda i, j: (0, j)),
            pl.BlockSpec((1, bn), lambda i, j: (0, j)),
            pl.BlockSpec((bm, bn), lambda i, j: (i, j)),
        ],
        out_specs=[
            pl.BlockSpec((bm, bn), lambda i, j: (i, j)),
            pl.BlockSpec((bm, bn), lambda i, j: (i, j)),
            pl.BlockSpec((bm, 1), lambda i, j: (i, 0)),
        ],
        out_shape=[
            jax.ShapeDtypeStruct((n, n), jnp.float32),
            jax.ShapeDtypeStruct((n, n), jnp.float32),
            jax.ShapeDtypeStruct((n, 1), jnp.float32),
        ],
        scratch_shapes=[
            pltpu.VMEM((1, nj * bn), jnp.float32),
            pltpu.VMEM((bm, 1), jnp.float32),
        ],
    )(sim, t, r, tT, rT, adj)
    return an, af, d2


def kernel(input, Adj, w_diag1, w_diag2, W1, b1, W2, b2):
    x = input
    n, feat = x.shape
    zf = jnp.zeros((1, feat), jnp.float32)

    deg = jnp.sum(Adj, axis=1)
    dinv = jnp.where(deg > 0, deg ** -0.5, 0.0)
    di = dinv[:, None]
    dj = dinv[None, :]
    c1 = _scale_cols(x, w_diag1[None, :])
    h = _gcn_mm(Adj, di, dj, c1, zf, "plain")
    c2 = _scale_cols(h, w_diag2[None, :])
    h2 = _gcn_mm(Adj, di, dj, c2, zf, "plain")
    nn = jnp.linalg.norm(h2, axis=1, keepdims=True)
    emb = h2 / jnp.maximum(nn, 1e-12)

    sim, t, r = _thresh(emb)
    tT = t.reshape(1, n)
    rT = r.reshape(1, n)

    an, af, d2 = _fuse(sim, t, r, tT, rT, Adj)
    dinv2 = jnp.where(d2[:, 0] > 0, d2[:, 0] ** -0.5, 0.0)
    di2 = dinv2[:, None]
    dj2 = dinv2[None, :]

    c3 = _dot(x, W1)
    h1 = _gcn_mm(af, di2, dj2, c3, b1[None, :], "relu_bias")
    c4 = _dot(h1, W2)
    out = _gcn_mm(af, di2, dj2, c4, b2[None, :], "bias")
    return out, an, af
